# baseline (speedup 1.0000x reference)
import functools

import jax
import jax.numpy as jnp
from jax import lax
from jax.experimental import pallas as pl
from jax.experimental.pallas import tpu as pltpu
from jax.experimental.pallas import tpu_sc as plsc

N = 10000
E = 320000
D = 128
H = 128
SL = 2
ES = E // SL
EB = 3200
NB = 400

NC = 2
NS = 16
L = 16
NW = NC * NS
CE = 128
NCHUNKS = ES // CE
FULL_TRIPS = NCHUNKS // NW
EXTRA = NCHUNKS - FULL_TRIPS * NW


def _silu(x):
    return x * jax.nn.sigmoid(x)


def _stage0_body(h_ref, wa_ref, wb_ref, p_ref, q_ref):
    hb = h_ref[...]
    p_ref[...] = jnp.dot(hb, wa_ref[...],
                         preferred_element_type=jnp.float32).astype(jnp.bfloat16)
    q_ref[...] = jnp.dot(hb, wb_ref[...],
                         preferred_element_type=jnp.float32).astype(jnp.bfloat16)


def _stage0(h, We1a, We1b):
    grid = (N // NB,)
    return pl.pallas_call(
        _stage0_body,
        grid=grid,
        in_specs=[
            pl.BlockSpec((NB, D), lambda i: (i, 0)),
            pl.BlockSpec((D, H), lambda i: (0, 0)),
            pl.BlockSpec((D, H), lambda i: (0, 0)),
        ],
        out_specs=[
            pl.BlockSpec((NB, H), lambda i: (i, 0)),
            pl.BlockSpec((NB, H), lambda i: (i, 0)),
        ],
        out_shape=[
            jax.ShapeDtypeStruct((N, H), jnp.bfloat16),
            jax.ShapeDtypeStruct((N, H), jnp.bfloat16),
        ],
    )(h, We1a, We1b)


def _sc_mesh():
    return plsc.VectorSubcoreMesh(core_axis_name="c", subcore_axis_name="s",
                                  num_cores=NC, num_subcores=NS)


def _stage1(P, Q, y3, row, col):
    @functools.partial(
        pl.kernel, mesh=_sc_mesh(),
        compiler_params=pltpu.CompilerParams(needs_layout_passes=False,
                                             use_tc_tiling_on_sc=False),
        out_type=[jax.ShapeDtypeStruct((ES, H), jnp.bfloat16),
                  jax.ShapeDtypeStruct((ES, H), jnp.bfloat16),
                  jax.ShapeDtypeStruct((4 * ES,), jnp.float32)],
        scratch_types=[
            pltpu.VMEM((CE,), jnp.int32),
            pltpu.VMEM((CE,), jnp.int32),
            pltpu.VMEM((CE,), jnp.int32),
            pltpu.VMEM((CE,), jnp.int32),
            pltpu.VMEM((CE, H), jnp.bfloat16),
            pltpu.VMEM((CE, H), jnp.bfloat16),
            pltpu.VMEM((CE, H), jnp.bfloat16),
            pltpu.VMEM((CE, H), jnp.bfloat16),
            pltpu.VMEM((4 * CE,), jnp.float32),
            pltpu.VMEM((4 * CE,), jnp.float32),
            pltpu.VMEM((3 * N,), jnp.float32),
            pltpu.SemaphoreType.DMA,
            pltpu.SemaphoreType.DMA,
            pltpu.SemaphoreType.DMA,
            pltpu.SemaphoreType.DMA,
        ],
    )
    def k1(p_hbm, q_hbm, y_hbm, row_hbm, col_hbm, sa_hbm, sb_hbm, cd_hbm,
           rv0, rv1, cv0, cv1, pb0, pb1, qb0, qb1, cd0, cd1, ybuf,
           semg0, semg1, semo0, semo1):
        wid = lax.axis_index("s") * NC + lax.axis_index("c")
        pltpu.sync_copy(y_hbm, ybuf)
        rv = [rv0, rv1]
        cv = [cv0, cv1]
        pb = [pb0, pb1]
        qb = [qb0, qb1]
        cdb = [cd0, cd1]
        semg = [semg0, semg1]
        semo = [semo0, semo1]
        iota = lax.iota(jnp.int32, L)
        ntrips = FULL_TRIPS + jnp.where(wid < EXTRA, 1, 0)

        def ebase(ci):
            return (wid + ci * NW) * CE

        def fetch(ci, b):
            base = ebase(ci)
            pltpu.sync_copy(row_hbm.at[pl.ds(base, CE)], rv[b])
            pltpu.sync_copy(col_hbm.at[pl.ds(base, CE)], cv[b])
            pltpu.async_copy(p_hbm.at[rv[b]], pb[b], semg[b])
            pltpu.async_copy(q_hbm.at[cv[b]], qb[b], semg[b])

        def wait_gathers(b):
            pltpu.make_async_copy(p_hbm.at[rv[b]], pb[b], semg[b]).wait()
            pltpu.make_async_copy(q_hbm.at[cv[b]], qb[b], semg[b]).wait()

        def wait_outs(ci, b):
            base = ebase(ci)
            pltpu.make_async_copy(pb[b], sa_hbm.at[pl.ds(base, CE)],
                                  semo[b]).wait()
            pltpu.make_async_copy(qb[b], sb_hbm.at[pl.ds(base, CE)],
                                  semo[b]).wait()
            pltpu.make_async_copy(cdb[b], cd_hbm.at[pl.ds(base * 4, CE * 4)],
                                  semo[b]).wait()

        def compute(ci, b):
            for g in range(CE // L):
                e16 = iota + g * L
                r16 = rv[b][pl.ds(g * L, L)]
                c16 = cv[b][pl.ds(g * L, L)]
                dx = (plsc.load_gather(ybuf, [r16])
                      - plsc.load_gather(ybuf, [c16]))
                dy = (plsc.load_gather(ybuf, [r16 + N])
                      - plsc.load_gather(ybuf, [c16 + N]))
                dz = (plsc.load_gather(ybuf, [r16 + 2 * N])
                      - plsc.load_gather(ybuf, [c16 + 2 * N]))
                e4 = e16 * 4
                plsc.store_scatter(cdb[b], [e4], dx)
                plsc.store_scatter(cdb[b], [e4 + 1], dy)
                plsc.store_scatter(cdb[b], [e4 + 2], dz)
                plsc.store_scatter(cdb[b], [e4 + 3],
                                   dx * dx + dy * dy + dz * dz)

            wait_gathers(b)
            base = ebase(ci)
            pltpu.async_copy(pb[b], sa_hbm.at[pl.ds(base, CE)], semo[b])
            pltpu.async_copy(qb[b], sb_hbm.at[pl.ds(base, CE)], semo[b])
            pltpu.async_copy(cdb[b], cd_hbm.at[pl.ds(base * 4, CE * 4)],
                             semo[b])

        def iteration(ci, b):
            nb = 1 - b

            @pl.when(ci + 1 < ntrips)
            def _():
                @pl.when(ci >= 1)
                def _():
                    wait_outs(ci - 1, nb)

                fetch(ci + 1, nb)

            @pl.when(ci < ntrips)
            def _():
                compute(ci, b)

        @pl.when(ntrips >= 1)
        def _():
            fetch(0, 0)

        def pair(j, carry):
            iteration(2 * j, 0)
            iteration(2 * j + 1, 1)
            return carry

        lax.fori_loop(0, (FULL_TRIPS + 2) // 2, pair, 0)
        wait_outs(ntrips - 1, 0)
        wait_outs(ntrips - 1, 1)

    return k1(P, Q, y3, row, col)


NPT = 624


def _stage3(feat, w1d, cd1, row):
    @functools.partial(
        pl.kernel, mesh=_sc_mesh(),
        compiler_params=pltpu.CompilerParams(needs_layout_passes=False,
                                             use_tc_tiling_on_sc=False),
        out_type=[jax.ShapeDtypeStruct((2 * N, H), jnp.float32),
                  jax.ShapeDtypeStruct((2 * N, 16), jnp.float32)],
        scratch_types=[
            pltpu.VMEM((CE,), jnp.int32),
            pltpu.VMEM((CE, H), jnp.float32),
            pltpu.VMEM((CE,), jnp.float32),
            pltpu.VMEM((4 * CE,), jnp.float32),
            pltpu.VMEM((CE, 16), jnp.float32),
            pltpu.VMEM_SHARED((N, H), jnp.float32),
            pltpu.VMEM_SHARED((N, 16), jnp.float32),
            pltpu.SemaphoreType.DMA,
            pltpu.SemaphoreType.DMA,
            pltpu.SemaphoreType.DMA,
        ],
    )
    def k3(f_hbm, w_hbm, cd_hbm, row_hbm, aggp_hbm, tp_hbm,
           row_v, fbuf, wb, cdb, tbuf, sh_agg, sh_t,
           sem0, sem1, sem2):
        cid = lax.axis_index("c")
        sid = lax.axis_index("s")
        wid = sid * NC + cid
        iota = lax.iota(jnp.int32, L)
        zeros = jnp.zeros((L,), jnp.float32)
        ones = jnp.ones((L,), jnp.float32)
        czero = jnp.zeros((L,), jnp.int32)
        cone = jnp.full((L,), 1, jnp.int32)
        ctwo = jnp.full((L,), 2, jnp.int32)
        cthree = jnp.full((L,), 3, jnp.int32)

        def zrow(i, carry):
            for k in range(H // L):
                fbuf[i, pl.ds(k * L, L)] = zeros
            tbuf[i, :] = zeros
            return carry

        lax.fori_loop(0, CE, zrow, 0)

        r0 = sid * NPT
        for j in range(4):
            pltpu.sync_copy(fbuf, sh_agg.at[pl.ds(r0 + j * CE, CE)])
            pltpu.sync_copy(tbuf, sh_t.at[pl.ds(r0 + j * CE, CE)])
        pltpu.sync_copy(fbuf.at[pl.ds(0, 112)],
                        sh_agg.at[pl.ds(r0 + 4 * CE, 112)])
        pltpu.sync_copy(tbuf.at[pl.ds(0, 112)],
                        sh_t.at[pl.ds(r0 + 4 * CE, 112)])

        @pl.when(sid == NS - 1)
        def _():
            pltpu.sync_copy(fbuf.at[pl.ds(0, 16)],
                            sh_agg.at[pl.ds(N - 16, 16)])
            pltpu.sync_copy(tbuf.at[pl.ds(0, 16)],
                            sh_t.at[pl.ds(N - 16, 16)])

        plsc.subcore_barrier()

        ntrips = FULL_TRIPS + jnp.where(wid < EXTRA, 1, 0)

        def chunk(i, carry):
            base = (wid + i * NW) * CE
            pltpu.sync_copy(row_hbm.at[pl.ds(base, CE)], row_v)
            cps = [pltpu.async_copy(f_hbm.at[pl.ds(base, CE)], fbuf, sem0),
                   pltpu.async_copy(w_hbm.at[pl.ds(base, CE)], wb, sem1),
                   pltpu.async_copy(cd_hbm.at[pl.ds(base * 4, CE * 4)],
                                    cdb, sem2)]
            for cp in cps:
                cp.wait()
            for g in range(CE // L):
                sl = pl.ds(g * L, L)
                e16 = iota + g * L
                e4 = e16 * 4
                wv = wb[sl]
                tx = plsc.load_gather(cdb, [e4]) * wv
                ty = plsc.load_gather(cdb, [e4 + 1]) * wv
                tz = plsc.load_gather(cdb, [e4 + 2]) * wv
                plsc.store_scatter(tbuf, [e16, czero], tx)
                plsc.store_scatter(tbuf, [e16, cone], ty)
                plsc.store_scatter(tbuf, [e16, ctwo], tz)
                plsc.store_scatter(tbuf, [e16, cthree], ones)
            pltpu.sync_copy(fbuf, sh_agg.at[row_v], add=True)
            pltpu.sync_copy(tbuf, sh_t.at[row_v], add=True)
            return carry

        lax.fori_loop(0, ntrips, chunk, 0)
        plsc.subcore_barrier()

        pltpu.sync_copy(sh_agg.at[pl.ds(r0, 624)],
                        aggp_hbm.at[pl.ds(cid * N + r0, 624)])
        pltpu.sync_copy(sh_t.at[pl.ds(r0, 624)],
                        tp_hbm.at[pl.ds(cid * N + r0, 624)])

        @pl.when(sid == NS - 1)
        def _():
            pltpu.sync_copy(sh_agg.at[pl.ds(N - 16, 16)],
                            aggp_hbm.at[pl.ds(cid * N + N - 16, 16)])
            pltpu.sync_copy(sh_t.at[pl.ds(N - 16, 16)],
                            tp_hbm.at[pl.ds(cid * N + N - 16, 16)])

    return k3(feat, w1d, cd1, row)


def _stage2_body(sa_ref, sb_ref, ea_ref, w1ea_ref, be1_ref, we2_ref, be2_ref,
                 wc1_ref, bc1_ref, wc2_ref, feat_ref, w_ref):
    s = (sa_ref[...].astype(jnp.float32)
         + sb_ref[...].astype(jnp.float32))
    ea = ea_ref[...]
    m = s + lax.dot_general(ea, w1ea_ref[...], (((0,), (0,)), ((), ())),
                            preferred_element_type=jnp.float32)
    m = _silu(m + be1_ref[...])
    feat = _silu(jnp.dot(m, we2_ref[...], preferred_element_type=jnp.float32)
                 + be2_ref[...])
    feat_ref[...] = feat
    c = _silu(jnp.dot(feat, wc1_ref[...], preferred_element_type=jnp.float32)
              + bc1_ref[...])
    w = lax.dot_general(wc2_ref[...], c, (((1,), (1,)), ((), ())),
                        preferred_element_type=jnp.float32)
    w_ref[...] = w[None]


def _stage2(sa, sb, ea5t, w1ea5, be1, We2, be2, Wc1, bc1, wc2row):
    grid = (ES // EB,)
    return pl.pallas_call(
        _stage2_body,
        grid=grid,
        in_specs=[
            pl.BlockSpec((EB, H), lambda i: (i, 0)),
            pl.BlockSpec((EB, H), lambda i: (i, 0)),
            pl.BlockSpec((5, EB), lambda i: (0, i)),
            pl.BlockSpec((5, H), lambda i: (0, 0)),
            pl.BlockSpec((1, H), lambda i: (0, 0)),
            pl.BlockSpec((H, H), lambda i: (0, 0)),
            pl.BlockSpec((1, H), lambda i: (0, 0)),
            pl.BlockSpec((H, H), lambda i: (0, 0)),
            pl.BlockSpec((1, H), lambda i: (0, 0)),
            pl.BlockSpec((1, H), lambda i: (0, 0)),
        ],
        out_specs=[
            pl.BlockSpec((EB, H), lambda i: (i, 0)),
            pl.BlockSpec((1, 1, EB), lambda i: (i, 0, 0)),
        ],
        out_shape=[
            jax.ShapeDtypeStruct((ES, H), jnp.float32),
            jax.ShapeDtypeStruct((ES // EB, 1, EB), jnp.float32),
        ],
    )(sa, sb, ea5t, w1ea5, be1, We2, be2, Wc1, bc1, wc2row)


def _stage4_body(h_ref, aggpa_ref, aggpb_ref, tpa_ref, tpb_ref, y_ref,
                 wn1h_ref, wn1a_ref, bn1_ref,
                 wn2_ref, bn2_ref, hnew_ref, ynew_ref):
    h = h_ref[...]
    agg = (aggpa_ref[0] + aggpa_ref[1]
           + aggpb_ref[0] + aggpb_ref[1])
    t = tpa_ref[0] + tpa_ref[1] + tpb_ref[0] + tpb_ref[1]
    u = _silu(jnp.dot(h, wn1h_ref[...], preferred_element_type=jnp.float32)
              + jnp.dot(agg, wn1a_ref[...], preferred_element_type=jnp.float32)
              + bn1_ref[...])
    hnew_ref[...] = h + jnp.dot(u, wn2_ref[...],
                                preferred_element_type=jnp.float32) + bn2_ref[...]
    cnt = jnp.maximum(t[:, 3:4], 1.0)
    ynew_ref[...] = y_ref[...] + t[:, 0:4] * jnp.concatenate(
        [jnp.ones((1, 3), jnp.float32), jnp.zeros((1, 1), jnp.float32)],
        axis=1) / cnt


def _stage4(h, aggpa, aggpb, tpa, tpb, y4, Wn1h, Wn1a, bn1, Wn2, bn2):
    grid = (N // NB,)
    return pl.pallas_call(
        _stage4_body,
        grid=grid,
        in_specs=[
            pl.BlockSpec((NB, D), lambda i: (i, 0)),
            pl.BlockSpec((2, NB, H), lambda i: (0, i, 0)),
            pl.BlockSpec((2, NB, H), lambda i: (0, i, 0)),
            pl.BlockSpec((2, NB, 16), lambda i: (0, i, 0)),
            pl.BlockSpec((2, NB, 16), lambda i: (0, i, 0)),
            pl.BlockSpec((NB, 4), lambda i: (i, 0)),
            pl.BlockSpec((D, H), lambda i: (0, 0)),
            pl.BlockSpec((H, H), lambda i: (0, 0)),
            pl.BlockSpec((1, H), lambda i: (0, 0)),
            pl.BlockSpec((H, D), lambda i: (0, 0)),
            pl.BlockSpec((1, D), lambda i: (0, 0)),
        ],
        out_specs=[
            pl.BlockSpec((NB, D), lambda i: (i, 0)),
            pl.BlockSpec((NB, 4), lambda i: (i, 0)),
        ],
        out_shape=[
            jax.ShapeDtypeStruct((N, D), jnp.float32),
            jax.ShapeDtypeStruct((N, 4), jnp.float32),
        ],
    )(h, aggpa, aggpb, tpa, tpb, y4, Wn1h, Wn1a, bn1, Wn2, bn2)


def kernel(h, edge_index, y, edge_attr, We1, be1, We2, be2,
           Wn1, bn1, Wn2, bn2, Wc1, bc1, Wc2):
    row = edge_index[0]
    col = edge_index[1]
    We1a = We1[:D]
    We1b = We1[D:2 * D]
    we1r = We1[2 * D]
    w1ea = We1[2 * D + 1:]

    P, Q = _stage0(h, We1a, We1b)

    y3 = y.T.reshape(3 * N)
    w1ea5 = jnp.concatenate([w1ea, we1r.reshape(1, H)], axis=0)

    aggps, tps = [], []
    for i in range(SL):
        sl = slice(i * ES, (i + 1) * ES)
        sa, sb, cd1 = _stage1(P, Q, y3, row[sl], col[sl])
        radial_row = cd1.reshape(ES, 4)[:, 3].reshape(1, ES)
        ea5t = jnp.concatenate([edge_attr[sl].T, radial_row], axis=0)
        feat, w2d = _stage2(sa, sb, ea5t, w1ea5, be1.reshape(1, H), We2,
                            be2.reshape(1, H), Wc1, bc1.reshape(1, H),
                            Wc2.reshape(1, H))
        aggp_flat, tp_flat = _stage3(feat, w2d.reshape(ES), cd1, row[sl])
        aggps.append(aggp_flat.reshape(2, N, H))
        tps.append(tp_flat.reshape(2, N, 16))

    y4 = jnp.pad(y, ((0, 0), (0, 1)))
    h_new, y_new4 = _stage4(h, aggps[0], aggps[1], tps[0], tps[1], y4,
                            Wn1[:D], Wn1[D:], bn1.reshape(1, H),
                            Wn2, bn2.reshape(1, D))
    return (h_new, y_new4[:, :3], edge_attr)


# speedup vs baseline: 1.6131x; 1.6131x over previous
import functools

import jax
import jax.numpy as jnp
from jax import lax
from jax.experimental import pallas as pl
from jax.experimental.pallas import tpu as pltpu
from jax.experimental.pallas import tpu_sc as plsc

N = 10000
E = 320000
D = 128
H = 128
SL = 2
ES = E // SL
EB = 3200
NB = 400

NC = 2
NS = 16
L = 16
NW = NC * NS
CE = 128
NCHUNKS = ES // CE
FULL_TRIPS = NCHUNKS // NW
EXTRA = NCHUNKS - FULL_TRIPS * NW


def _silu(x):
    return x * jax.nn.sigmoid(x)


def _stage0_body(h_ref, wa_ref, wb_ref, p_ref, q_ref):
    hb = h_ref[...]
    p_ref[...] = jnp.dot(hb, wa_ref[...], preferred_element_type=jnp.float32)
    q_ref[...] = jnp.dot(hb, wb_ref[...], preferred_element_type=jnp.float32)


def _stage0(h, We1a, We1b):
    grid = (N // NB,)
    return pl.pallas_call(
        _stage0_body,
        grid=grid,
        in_specs=[
            pl.BlockSpec((NB, D), lambda i: (i, 0)),
            pl.BlockSpec((D, H), lambda i: (0, 0)),
            pl.BlockSpec((D, H), lambda i: (0, 0)),
        ],
        out_specs=[
            pl.BlockSpec((NB, H), lambda i: (i, 0)),
            pl.BlockSpec((NB, H), lambda i: (i, 0)),
        ],
        out_shape=[
            jax.ShapeDtypeStruct((N, H), jnp.float32),
            jax.ShapeDtypeStruct((N, H), jnp.float32),
        ],
    )(h, We1a, We1b)


def _sc_mesh():
    return plsc.VectorSubcoreMesh(core_axis_name="c", subcore_axis_name="s",
                                  num_cores=NC, num_subcores=NS)


def _stage1(P, Q, y3, row, col):
    @functools.partial(
        pl.kernel, mesh=_sc_mesh(),
        compiler_params=pltpu.CompilerParams(needs_layout_passes=False,
                                             use_tc_tiling_on_sc=False),
        out_type=[jax.ShapeDtypeStruct((ES, H), jnp.float32),
                  jax.ShapeDtypeStruct((ES, H), jnp.float32),
                  jax.ShapeDtypeStruct((4 * ES,), jnp.float32)],
        scratch_types=[
            pltpu.VMEM((CE,), jnp.int32),
            pltpu.VMEM((CE,), jnp.int32),
            pltpu.VMEM((CE,), jnp.int32),
            pltpu.VMEM((CE,), jnp.int32),
            pltpu.VMEM((CE, H), jnp.float32),
            pltpu.VMEM((CE, H), jnp.float32),
            pltpu.VMEM((CE, H), jnp.float32),
            pltpu.VMEM((CE, H), jnp.float32),
            pltpu.VMEM((4 * CE,), jnp.float32),
            pltpu.VMEM((4 * CE,), jnp.float32),
            pltpu.VMEM((3 * N,), jnp.float32),
            pltpu.SemaphoreType.DMA,
            pltpu.SemaphoreType.DMA,
            pltpu.SemaphoreType.DMA,
            pltpu.SemaphoreType.DMA,
        ],
    )
    def k1(p_hbm, q_hbm, y_hbm, row_hbm, col_hbm, sa_hbm, sb_hbm, cd_hbm,
           rv0, rv1, cv0, cv1, pb0, pb1, qb0, qb1, cd0, cd1, ybuf,
           semg0, semg1, semo0, semo1):
        wid = lax.axis_index("s") * NC + lax.axis_index("c")
        pltpu.sync_copy(y_hbm, ybuf)
        rv = [rv0, rv1]
        cv = [cv0, cv1]
        pb = [pb0, pb1]
        qb = [qb0, qb1]
        cdb = [cd0, cd1]
        semg = [semg0, semg1]
        semo = [semo0, semo1]
        iota = lax.iota(jnp.int32, L)
        ntrips = FULL_TRIPS + jnp.where(wid < EXTRA, 1, 0)

        def ebase(ci):
            return (wid + ci * NW) * CE

        def fetch(ci, b):
            base = ebase(ci)
            pltpu.sync_copy(row_hbm.at[pl.ds(base, CE)], rv[b])
            pltpu.sync_copy(col_hbm.at[pl.ds(base, CE)], cv[b])
            pltpu.async_copy(p_hbm.at[rv[b]], pb[b], semg[b])
            pltpu.async_copy(q_hbm.at[cv[b]], qb[b], semg[b])

        def wait_gathers(b):
            pltpu.make_async_copy(p_hbm.at[rv[b]], pb[b], semg[b]).wait()
            pltpu.make_async_copy(q_hbm.at[cv[b]], qb[b], semg[b]).wait()

        def wait_outs(ci, b):
            base = ebase(ci)
            pltpu.make_async_copy(pb[b], sa_hbm.at[pl.ds(base, CE)],
                                  semo[b]).wait()
            pltpu.make_async_copy(qb[b], sb_hbm.at[pl.ds(base, CE)],
                                  semo[b]).wait()
            pltpu.make_async_copy(cdb[b], cd_hbm.at[pl.ds(base * 4, CE * 4)],
                                  semo[b]).wait()

        def compute(ci, b):
            for g in range(CE // L):
                e16 = iota + g * L
                r16 = rv[b][pl.ds(g * L, L)]
                c16 = cv[b][pl.ds(g * L, L)]
                dx = (plsc.load_gather(ybuf, [r16])
                      - plsc.load_gather(ybuf, [c16]))
                dy = (plsc.load_gather(ybuf, [r16 + N])
                      - plsc.load_gather(ybuf, [c16 + N]))
                dz = (plsc.load_gather(ybuf, [r16 + 2 * N])
                      - plsc.load_gather(ybuf, [c16 + 2 * N]))
                e4 = e16 * 4
                plsc.store_scatter(cdb[b], [e4], dx)
                plsc.store_scatter(cdb[b], [e4 + 1], dy)
                plsc.store_scatter(cdb[b], [e4 + 2], dz)
                plsc.store_scatter(cdb[b], [e4 + 3],
                                   dx * dx + dy * dy + dz * dz)

            wait_gathers(b)
            base = ebase(ci)
            pltpu.async_copy(pb[b], sa_hbm.at[pl.ds(base, CE)], semo[b])
            pltpu.async_copy(qb[b], sb_hbm.at[pl.ds(base, CE)], semo[b])
            pltpu.async_copy(cdb[b], cd_hbm.at[pl.ds(base * 4, CE * 4)],
                             semo[b])

        def iteration(ci, b):
            nb = 1 - b

            @pl.when(ci + 1 < ntrips)
            def _():
                @pl.when(ci >= 1)
                def _():
                    wait_outs(ci - 1, nb)

                fetch(ci + 1, nb)

            @pl.when(ci < ntrips)
            def _():
                compute(ci, b)

        @pl.when(ntrips >= 1)
        def _():
            fetch(0, 0)

        def pair(j, carry):
            iteration(2 * j, 0)
            iteration(2 * j + 1, 1)
            return carry

        lax.fori_loop(0, (FULL_TRIPS + 2) // 2, pair, 0)
        wait_outs(ntrips - 1, 0)
        wait_outs(ntrips - 1, 1)

    return k1(P, Q, y3, row, col)


NPT = 624


def _stage3(feat, w1d, cd1, row):
    @functools.partial(
        pl.kernel, mesh=_sc_mesh(),
        compiler_params=pltpu.CompilerParams(needs_layout_passes=False,
                                             use_tc_tiling_on_sc=False),
        out_type=[jax.ShapeDtypeStruct((2 * N, H), jnp.float32),
                  jax.ShapeDtypeStruct((2 * N, 16), jnp.float32)],
        scratch_types=[
            pltpu.VMEM((CE,), jnp.int32),
            pltpu.VMEM((CE, H), jnp.float32),
            pltpu.VMEM((CE,), jnp.float32),
            pltpu.VMEM((4 * CE,), jnp.float32),
            pltpu.VMEM((CE, 16), jnp.float32),
            pltpu.VMEM_SHARED((N, H), jnp.float32),
            pltpu.VMEM_SHARED((N, 16), jnp.float32),
            pltpu.SemaphoreType.DMA,
            pltpu.SemaphoreType.DMA,
            pltpu.SemaphoreType.DMA,
        ],
    )
    def k3(f_hbm, w_hbm, cd_hbm, row_hbm, aggp_hbm, tp_hbm,
           row_v, fbuf, wb, cdb, tbuf, sh_agg, sh_t,
           sem0, sem1, sem2):
        cid = lax.axis_index("c")
        sid = lax.axis_index("s")
        wid = sid * NC + cid
        iota = lax.iota(jnp.int32, L)
        zeros = jnp.zeros((L,), jnp.float32)
        ones = jnp.ones((L,), jnp.float32)
        czero = jnp.zeros((L,), jnp.int32)
        cone = jnp.full((L,), 1, jnp.int32)
        ctwo = jnp.full((L,), 2, jnp.int32)
        cthree = jnp.full((L,), 3, jnp.int32)

        def zrow(i, carry):
            for k in range(H // L):
                fbuf[i, pl.ds(k * L, L)] = zeros
            tbuf[i, :] = zeros
            return carry

        lax.fori_loop(0, CE, zrow, 0)

        r0 = sid * NPT
        for j in range(4):
            pltpu.sync_copy(fbuf, sh_agg.at[pl.ds(r0 + j * CE, CE)])
            pltpu.sync_copy(tbuf, sh_t.at[pl.ds(r0 + j * CE, CE)])
        pltpu.sync_copy(fbuf.at[pl.ds(0, 112)],
                        sh_agg.at[pl.ds(r0 + 4 * CE, 112)])
        pltpu.sync_copy(tbuf.at[pl.ds(0, 112)],
                        sh_t.at[pl.ds(r0 + 4 * CE, 112)])

        @pl.when(sid == NS - 1)
        def _():
            pltpu.sync_copy(fbuf.at[pl.ds(0, 16)],
                            sh_agg.at[pl.ds(N - 16, 16)])
            pltpu.sync_copy(tbuf.at[pl.ds(0, 16)],
                            sh_t.at[pl.ds(N - 16, 16)])

        plsc.subcore_barrier()

        ntrips = FULL_TRIPS + jnp.where(wid < EXTRA, 1, 0)

        def chunk(i, carry):
            base = (wid + i * NW) * CE
            pltpu.sync_copy(row_hbm.at[pl.ds(base, CE)], row_v)
            cps = [pltpu.async_copy(f_hbm.at[pl.ds(base, CE)], fbuf, sem0),
                   pltpu.async_copy(w_hbm.at[pl.ds(base, CE)], wb, sem1),
                   pltpu.async_copy(cd_hbm.at[pl.ds(base * 4, CE * 4)],
                                    cdb, sem2)]
            for cp in cps:
                cp.wait()
            for g in range(CE // L):
                sl = pl.ds(g * L, L)
                e16 = iota + g * L
                e4 = e16 * 4
                wv = wb[sl]
                tx = plsc.load_gather(cdb, [e4]) * wv
                ty = plsc.load_gather(cdb, [e4 + 1]) * wv
                tz = plsc.load_gather(cdb, [e4 + 2]) * wv
                plsc.store_scatter(tbuf, [e16, czero], tx)
                plsc.store_scatter(tbuf, [e16, cone], ty)
                plsc.store_scatter(tbuf, [e16, ctwo], tz)
                plsc.store_scatter(tbuf, [e16, cthree], ones)
            pltpu.sync_copy(fbuf, sh_agg.at[row_v], add=True)
            pltpu.sync_copy(tbuf, sh_t.at[row_v], add=True)
            return carry

        lax.fori_loop(0, ntrips, chunk, 0)
        plsc.subcore_barrier()

        pltpu.sync_copy(sh_agg.at[pl.ds(r0, 624)],
                        aggp_hbm.at[pl.ds(cid * N + r0, 624)])
        pltpu.sync_copy(sh_t.at[pl.ds(r0, 624)],
                        tp_hbm.at[pl.ds(cid * N + r0, 624)])

        @pl.when(sid == NS - 1)
        def _():
            pltpu.sync_copy(sh_agg.at[pl.ds(N - 16, 16)],
                            aggp_hbm.at[pl.ds(cid * N + N - 16, 16)])
            pltpu.sync_copy(sh_t.at[pl.ds(N - 16, 16)],
                            tp_hbm.at[pl.ds(cid * N + N - 16, 16)])

    return k3(feat, w1d, cd1, row)


def _stage2_body(sa_ref, sb_ref, ea_ref, w1ea_ref, be1_ref, we2_ref, be2_ref,
                 wc1_ref, bc1_ref, wc2_ref, feat_ref, w_ref):
    s = sa_ref[...] + sb_ref[...]
    ea = ea_ref[...]
    m = s + lax.dot_general(ea, w1ea_ref[...], (((0,), (0,)), ((), ())),
                            preferred_element_type=jnp.float32)
    m = _silu(m + be1_ref[...])
    feat = _silu(jnp.dot(m, we2_ref[...], preferred_element_type=jnp.float32)
                 + be2_ref[...])
    feat_ref[...] = feat
    c = _silu(jnp.dot(feat, wc1_ref[...], preferred_element_type=jnp.float32)
              + bc1_ref[...])
    w = lax.dot_general(wc2_ref[...], c, (((1,), (1,)), ((), ())),
                        preferred_element_type=jnp.float32)
    w_ref[...] = w[None]


def _stage2(sa, sb, ea5t, w1ea5, be1, We2, be2, Wc1, bc1, wc2row):
    grid = (ES // EB,)
    return pl.pallas_call(
        _stage2_body,
        grid=grid,
        in_specs=[
            pl.BlockSpec((EB, H), lambda i: (i, 0)),
            pl.BlockSpec((EB, H), lambda i: (i, 0)),
            pl.BlockSpec((5, EB), lambda i: (0, i)),
            pl.BlockSpec((5, H), lambda i: (0, 0)),
            pl.BlockSpec((1, H), lambda i: (0, 0)),
            pl.BlockSpec((H, H), lambda i: (0, 0)),
            pl.BlockSpec((1, H), lambda i: (0, 0)),
            pl.BlockSpec((H, H), lambda i: (0, 0)),
            pl.BlockSpec((1, H), lambda i: (0, 0)),
            pl.BlockSpec((1, H), lambda i: (0, 0)),
        ],
        out_specs=[
            pl.BlockSpec((EB, H), lambda i: (i, 0)),
            pl.BlockSpec((1, 1, EB), lambda i: (i, 0, 0)),
        ],
        out_shape=[
            jax.ShapeDtypeStruct((ES, H), jnp.float32),
            jax.ShapeDtypeStruct((ES // EB, 1, EB), jnp.float32),
        ],
    )(sa, sb, ea5t, w1ea5, be1, We2, be2, Wc1, bc1, wc2row)


def _stage4_body(h_ref, aggpa_ref, aggpb_ref, tpa_ref, tpb_ref, y_ref,
                 wn1h_ref, wn1a_ref, bn1_ref,
                 wn2_ref, bn2_ref, hnew_ref, ynew_ref):
    h = h_ref[...]
    agg = (aggpa_ref[0] + aggpa_ref[1]
           + aggpb_ref[0] + aggpb_ref[1])
    t = tpa_ref[0] + tpa_ref[1] + tpb_ref[0] + tpb_ref[1]
    u = _silu(jnp.dot(h, wn1h_ref[...], preferred_element_type=jnp.float32)
              + jnp.dot(agg, wn1a_ref[...], preferred_element_type=jnp.float32)
              + bn1_ref[...])
    hnew_ref[...] = h + jnp.dot(u, wn2_ref[...],
                                preferred_element_type=jnp.float32) + bn2_ref[...]
    cnt = jnp.maximum(t[:, 3:4], 1.0)
    ynew_ref[...] = y_ref[...] + t[:, 0:4] * jnp.concatenate(
        [jnp.ones((1, 3), jnp.float32), jnp.zeros((1, 1), jnp.float32)],
        axis=1) / cnt


def _stage4(h, aggpa, aggpb, tpa, tpb, y4, Wn1h, Wn1a, bn1, Wn2, bn2):
    grid = (N // NB,)
    return pl.pallas_call(
        _stage4_body,
        grid=grid,
        in_specs=[
            pl.BlockSpec((NB, D), lambda i: (i, 0)),
            pl.BlockSpec((2, NB, H), lambda i: (0, i, 0)),
            pl.BlockSpec((2, NB, H), lambda i: (0, i, 0)),
            pl.BlockSpec((2, NB, 16), lambda i: (0, i, 0)),
            pl.BlockSpec((2, NB, 16), lambda i: (0, i, 0)),
            pl.BlockSpec((NB, 4), lambda i: (i, 0)),
            pl.BlockSpec((D, H), lambda i: (0, 0)),
            pl.BlockSpec((H, H), lambda i: (0, 0)),
            pl.BlockSpec((1, H), lambda i: (0, 0)),
            pl.BlockSpec((H, D), lambda i: (0, 0)),
            pl.BlockSpec((1, D), lambda i: (0, 0)),
        ],
        out_specs=[
            pl.BlockSpec((NB, D), lambda i: (i, 0)),
            pl.BlockSpec((NB, 4), lambda i: (i, 0)),
        ],
        out_shape=[
            jax.ShapeDtypeStruct((N, D), jnp.float32),
            jax.ShapeDtypeStruct((N, 4), jnp.float32),
        ],
    )(h, aggpa, aggpb, tpa, tpb, y4, Wn1h, Wn1a, bn1, Wn2, bn2)


def kernel(h, edge_index, y, edge_attr, We1, be1, We2, be2,
           Wn1, bn1, Wn2, bn2, Wc1, bc1, Wc2):
    row = edge_index[0]
    col = edge_index[1]
    We1a = We1[:D]
    We1b = We1[D:2 * D]
    we1r = We1[2 * D]
    w1ea = We1[2 * D + 1:]

    P, Q = _stage0(h, We1a, We1b)

    y3 = y.T.reshape(3 * N)
    w1ea5 = jnp.concatenate([w1ea, we1r.reshape(1, H)], axis=0)

    aggps, tps = [], []
    for i in range(SL):
        sl = slice(i * ES, (i + 1) * ES)
        sa, sb, cd1 = _stage1(P, Q, y3, row[sl], col[sl])
        radial_row = cd1.reshape(ES, 4)[:, 3].reshape(1, ES)
        ea5t = jnp.concatenate([edge_attr[sl].T, radial_row], axis=0)
        feat, w2d = _stage2(sa, sb, ea5t, w1ea5, be1.reshape(1, H), We2,
                            be2.reshape(1, H), Wc1, bc1.reshape(1, H),
                            Wc2.reshape(1, H))
        aggp_flat, tp_flat = _stage3(feat, w2d.reshape(ES), cd1, row[sl])
        aggps.append(aggp_flat.reshape(2, N, H))
        tps.append(tp_flat.reshape(2, N, 16))

    y4 = jnp.pad(y, ((0, 0), (0, 1)))
    h_new, y_new4 = _stage4(h, aggps[0], aggps[1], tps[0], tps[1], y4,
                            Wn1[:D], Wn1[D:], bn1.reshape(1, H),
                            Wn2, bn2.reshape(1, D))
    return (h_new, y_new4[:, :3], edge_attr)


# speedup vs baseline: 1.6410x; 1.0173x over previous
import functools

import jax
import jax.numpy as jnp
from jax import lax
from jax.experimental import pallas as pl
from jax.experimental.pallas import tpu as pltpu
from jax.experimental.pallas import tpu_sc as plsc

N = 10000
E = 320000
D = 128
H = 128
SL = 4
ES = E // SL
EB = 3200
NB = 400

NC = 2
NS = 16
L = 16
NW = NC * NS
CE = 128
NCHUNKS = ES // CE
FULL_TRIPS = NCHUNKS // NW
EXTRA = NCHUNKS - FULL_TRIPS * NW


def _silu(x):
    return x * jax.nn.sigmoid(x)


def _stage0_body(h_ref, wa_ref, wb_ref, p_ref, q_ref):
    hb = h_ref[...]
    p_ref[...] = jnp.dot(hb, wa_ref[...], preferred_element_type=jnp.float32)
    q_ref[...] = jnp.dot(hb, wb_ref[...], preferred_element_type=jnp.float32)


def _stage0(h, We1a, We1b):
    grid = (N // NB,)
    return pl.pallas_call(
        _stage0_body,
        grid=grid,
        in_specs=[
            pl.BlockSpec((NB, D), lambda i: (i, 0)),
            pl.BlockSpec((D, H), lambda i: (0, 0)),
            pl.BlockSpec((D, H), lambda i: (0, 0)),
        ],
        out_specs=[
            pl.BlockSpec((NB, H), lambda i: (i, 0)),
            pl.BlockSpec((NB, H), lambda i: (i, 0)),
        ],
        out_shape=[
            jax.ShapeDtypeStruct((N, H), jnp.float32),
            jax.ShapeDtypeStruct((N, H), jnp.float32),
        ],
    )(h, We1a, We1b)


def _sc_mesh():
    return plsc.VectorSubcoreMesh(core_axis_name="c", subcore_axis_name="s",
                                  num_cores=NC, num_subcores=NS)


def _stage1(P, Q, y3, row, col):
    @functools.partial(
        pl.kernel, mesh=_sc_mesh(),
        compiler_params=pltpu.CompilerParams(needs_layout_passes=False,
                                             use_tc_tiling_on_sc=False),
        out_type=[jax.ShapeDtypeStruct((ES, H), jnp.float32),
                  jax.ShapeDtypeStruct((ES, H), jnp.float32),
                  jax.ShapeDtypeStruct((4 * ES,), jnp.float32)],
        scratch_types=[
            pltpu.VMEM((CE,), jnp.int32),
            pltpu.VMEM((CE,), jnp.int32),
            pltpu.VMEM((CE,), jnp.int32),
            pltpu.VMEM((CE,), jnp.int32),
            pltpu.VMEM((CE, H), jnp.float32),
            pltpu.VMEM((CE, H), jnp.float32),
            pltpu.VMEM((CE, H), jnp.float32),
            pltpu.VMEM((CE, H), jnp.float32),
            pltpu.VMEM((4 * CE,), jnp.float32),
            pltpu.VMEM((4 * CE,), jnp.float32),
            pltpu.VMEM((3 * N,), jnp.float32),
            pltpu.SemaphoreType.DMA,
            pltpu.SemaphoreType.DMA,
            pltpu.SemaphoreType.DMA,
            pltpu.SemaphoreType.DMA,
        ],
    )
    def k1(p_hbm, q_hbm, y_hbm, row_hbm, col_hbm, sa_hbm, sb_hbm, cd_hbm,
           rv0, rv1, cv0, cv1, pb0, pb1, qb0, qb1, cd0, cd1, ybuf,
           semg0, semg1, semo0, semo1):
        wid = lax.axis_index("s") * NC + lax.axis_index("c")
        pltpu.sync_copy(y_hbm, ybuf)
        rv = [rv0, rv1]
        cv = [cv0, cv1]
        pb = [pb0, pb1]
        qb = [qb0, qb1]
        cdb = [cd0, cd1]
        semg = [semg0, semg1]
        semo = [semo0, semo1]
        iota = lax.iota(jnp.int32, L)
        ntrips = FULL_TRIPS + jnp.where(wid < EXTRA, 1, 0)

        def ebase(ci):
            return (wid + ci * NW) * CE

        def fetch(ci, b):
            base = ebase(ci)
            pltpu.sync_copy(row_hbm.at[pl.ds(base, CE)], rv[b])
            pltpu.sync_copy(col_hbm.at[pl.ds(base, CE)], cv[b])
            pltpu.async_copy(p_hbm.at[rv[b]], pb[b], semg[b])
            pltpu.async_copy(q_hbm.at[cv[b]], qb[b], semg[b])

        def wait_gathers(b):
            pltpu.make_async_copy(p_hbm.at[rv[b]], pb[b], semg[b]).wait()
            pltpu.make_async_copy(q_hbm.at[cv[b]], qb[b], semg[b]).wait()

        def wait_outs(ci, b):
            base = ebase(ci)
            pltpu.make_async_copy(pb[b], sa_hbm.at[pl.ds(base, CE)],
                                  semo[b]).wait()
            pltpu.make_async_copy(qb[b], sb_hbm.at[pl.ds(base, CE)],
                                  semo[b]).wait()
            pltpu.make_async_copy(cdb[b], cd_hbm.at[pl.ds(base * 4, CE * 4)],
                                  semo[b]).wait()

        def compute(ci, b):
            for g in range(CE // L):
                e16 = iota + g * L
                r16 = rv[b][pl.ds(g * L, L)]
                c16 = cv[b][pl.ds(g * L, L)]
                dx = (plsc.load_gather(ybuf, [r16])
                      - plsc.load_gather(ybuf, [c16]))
                dy = (plsc.load_gather(ybuf, [r16 + N])
                      - plsc.load_gather(ybuf, [c16 + N]))
                dz = (plsc.load_gather(ybuf, [r16 + 2 * N])
                      - plsc.load_gather(ybuf, [c16 + 2 * N]))
                e4 = e16 * 4
                plsc.store_scatter(cdb[b], [e4], dx)
                plsc.store_scatter(cdb[b], [e4 + 1], dy)
                plsc.store_scatter(cdb[b], [e4 + 2], dz)
                plsc.store_scatter(cdb[b], [e4 + 3],
                                   dx * dx + dy * dy + dz * dz)

            wait_gathers(b)
            base = ebase(ci)
            pltpu.async_copy(pb[b], sa_hbm.at[pl.ds(base, CE)], semo[b])
            pltpu.async_copy(qb[b], sb_hbm.at[pl.ds(base, CE)], semo[b])
            pltpu.async_copy(cdb[b], cd_hbm.at[pl.ds(base * 4, CE * 4)],
                             semo[b])

        def iteration(ci, b):
            nb = 1 - b

            @pl.when(ci + 1 < ntrips)
            def _():
                @pl.when(ci >= 1)
                def _():
                    wait_outs(ci - 1, nb)

                fetch(ci + 1, nb)

            @pl.when(ci < ntrips)
            def _():
                compute(ci, b)

        @pl.when(ntrips >= 1)
        def _():
            fetch(0, 0)

        def pair(j, carry):
            iteration(2 * j, 0)
            iteration(2 * j + 1, 1)
            return carry

        lax.fori_loop(0, (FULL_TRIPS + 2) // 2, pair, 0)
        wait_outs(ntrips - 1, 0)
        wait_outs(ntrips - 1, 1)

    return k1(P, Q, y3, row, col)


NPT = 624


def _stage3(feat, w1d, cd1, row):
    @functools.partial(
        pl.kernel, mesh=_sc_mesh(),
        compiler_params=pltpu.CompilerParams(needs_layout_passes=False,
                                             use_tc_tiling_on_sc=False),
        out_type=[jax.ShapeDtypeStruct((2 * N, H), jnp.float32),
                  jax.ShapeDtypeStruct((2 * N, 16), jnp.float32)],
        scratch_types=[
            pltpu.VMEM((CE,), jnp.int32),
            pltpu.VMEM((CE, H), jnp.float32),
            pltpu.VMEM((CE,), jnp.float32),
            pltpu.VMEM((4 * CE,), jnp.float32),
            pltpu.VMEM((CE, 16), jnp.float32),
            pltpu.VMEM_SHARED((N, H), jnp.float32),
            pltpu.VMEM_SHARED((N, 16), jnp.float32),
            pltpu.SemaphoreType.DMA,
            pltpu.SemaphoreType.DMA,
            pltpu.SemaphoreType.DMA,
        ],
    )
    def k3(f_hbm, w_hbm, cd_hbm, row_hbm, aggp_hbm, tp_hbm,
           row_v, fbuf, wb, cdb, tbuf, sh_agg, sh_t,
           sem0, sem1, sem2):
        cid = lax.axis_index("c")
        sid = lax.axis_index("s")
        wid = sid * NC + cid
        iota = lax.iota(jnp.int32, L)
        zeros = jnp.zeros((L,), jnp.float32)
        ones = jnp.ones((L,), jnp.float32)
        czero = jnp.zeros((L,), jnp.int32)
        cone = jnp.full((L,), 1, jnp.int32)
        ctwo = jnp.full((L,), 2, jnp.int32)
        cthree = jnp.full((L,), 3, jnp.int32)

        def zrow(i, carry):
            for k in range(H // L):
                fbuf[i, pl.ds(k * L, L)] = zeros
            tbuf[i, :] = zeros
            return carry

        lax.fori_loop(0, CE, zrow, 0)

        r0 = sid * NPT
        for j in range(4):
            pltpu.sync_copy(fbuf, sh_agg.at[pl.ds(r0 + j * CE, CE)])
            pltpu.sync_copy(tbuf, sh_t.at[pl.ds(r0 + j * CE, CE)])
        pltpu.sync_copy(fbuf.at[pl.ds(0, 112)],
                        sh_agg.at[pl.ds(r0 + 4 * CE, 112)])
        pltpu.sync_copy(tbuf.at[pl.ds(0, 112)],
                        sh_t.at[pl.ds(r0 + 4 * CE, 112)])

        @pl.when(sid == NS - 1)
        def _():
            pltpu.sync_copy(fbuf.at[pl.ds(0, 16)],
                            sh_agg.at[pl.ds(N - 16, 16)])
            pltpu.sync_copy(tbuf.at[pl.ds(0, 16)],
                            sh_t.at[pl.ds(N - 16, 16)])

        plsc.subcore_barrier()

        ntrips = FULL_TRIPS + jnp.where(wid < EXTRA, 1, 0)

        def chunk(i, carry):
            base = (wid + i * NW) * CE
            pltpu.sync_copy(row_hbm.at[pl.ds(base, CE)], row_v)
            cps = [pltpu.async_copy(f_hbm.at[pl.ds(base, CE)], fbuf, sem0),
                   pltpu.async_copy(w_hbm.at[pl.ds(base, CE)], wb, sem1),
                   pltpu.async_copy(cd_hbm.at[pl.ds(base * 4, CE * 4)],
                                    cdb, sem2)]
            for cp in cps:
                cp.wait()
            for g in range(CE // L):
                sl = pl.ds(g * L, L)
                e16 = iota + g * L
                e4 = e16 * 4
                wv = wb[sl]
                tx = plsc.load_gather(cdb, [e4]) * wv
                ty = plsc.load_gather(cdb, [e4 + 1]) * wv
                tz = plsc.load_gather(cdb, [e4 + 2]) * wv
                plsc.store_scatter(tbuf, [e16, czero], tx)
                plsc.store_scatter(tbuf, [e16, cone], ty)
                plsc.store_scatter(tbuf, [e16, ctwo], tz)
                plsc.store_scatter(tbuf, [e16, cthree], ones)
            pltpu.sync_copy(fbuf, sh_agg.at[row_v], add=True)
            pltpu.sync_copy(tbuf, sh_t.at[row_v], add=True)
            return carry

        lax.fori_loop(0, ntrips, chunk, 0)
        plsc.subcore_barrier()

        pltpu.sync_copy(sh_agg.at[pl.ds(r0, 624)],
                        aggp_hbm.at[pl.ds(cid * N + r0, 624)])
        pltpu.sync_copy(sh_t.at[pl.ds(r0, 624)],
                        tp_hbm.at[pl.ds(cid * N + r0, 624)])

        @pl.when(sid == NS - 1)
        def _():
            pltpu.sync_copy(sh_agg.at[pl.ds(N - 16, 16)],
                            aggp_hbm.at[pl.ds(cid * N + N - 16, 16)])
            pltpu.sync_copy(sh_t.at[pl.ds(N - 16, 16)],
                            tp_hbm.at[pl.ds(cid * N + N - 16, 16)])

    return k3(feat, w1d, cd1, row)


def _stage2_body(sa_ref, sb_ref, ea_ref, w1ea_ref, be1_ref, we2_ref, be2_ref,
                 wc1_ref, bc1_ref, wc2_ref, feat_ref, w_ref):
    s = sa_ref[...] + sb_ref[...]
    ea = ea_ref[...]
    m = s + lax.dot_general(ea, w1ea_ref[...], (((0,), (0,)), ((), ())),
                            preferred_element_type=jnp.float32)
    m = _silu(m + be1_ref[...])
    feat = _silu(jnp.dot(m, we2_ref[...], preferred_element_type=jnp.float32)
                 + be2_ref[...])
    feat_ref[...] = feat
    c = _silu(jnp.dot(feat, wc1_ref[...], preferred_element_type=jnp.float32)
              + bc1_ref[...])
    w = lax.dot_general(wc2_ref[...], c, (((1,), (1,)), ((), ())),
                        preferred_element_type=jnp.float32)
    w_ref[...] = w[None]


def _stage2(sa, sb, ea5t, w1ea5, be1, We2, be2, Wc1, bc1, wc2row):
    grid = (ES // EB,)
    return pl.pallas_call(
        _stage2_body,
        grid=grid,
        in_specs=[
            pl.BlockSpec((EB, H), lambda i: (i, 0)),
            pl.BlockSpec((EB, H), lambda i: (i, 0)),
            pl.BlockSpec((5, EB), lambda i: (0, i)),
            pl.BlockSpec((5, H), lambda i: (0, 0)),
            pl.BlockSpec((1, H), lambda i: (0, 0)),
            pl.BlockSpec((H, H), lambda i: (0, 0)),
            pl.BlockSpec((1, H), lambda i: (0, 0)),
            pl.BlockSpec((H, H), lambda i: (0, 0)),
            pl.BlockSpec((1, H), lambda i: (0, 0)),
            pl.BlockSpec((1, H), lambda i: (0, 0)),
        ],
        out_specs=[
            pl.BlockSpec((EB, H), lambda i: (i, 0)),
            pl.BlockSpec((1, 1, EB), lambda i: (i, 0, 0)),
        ],
        out_shape=[
            jax.ShapeDtypeStruct((ES, H), jnp.float32),
            jax.ShapeDtypeStruct((ES // EB, 1, EB), jnp.float32),
        ],
    )(sa, sb, ea5t, w1ea5, be1, We2, be2, Wc1, bc1, wc2row)


def _stage4_body(h_ref, aggp_ref, tp_ref, y_ref,
                 wn1h_ref, wn1a_ref, bn1_ref,
                 wn2_ref, bn2_ref, hnew_ref, ynew_ref):
    h = h_ref[...]
    agg = jnp.sum(aggp_ref[...], axis=0)
    t = jnp.sum(tp_ref[...], axis=0)
    u = _silu(jnp.dot(h, wn1h_ref[...], preferred_element_type=jnp.float32)
              + jnp.dot(agg, wn1a_ref[...], preferred_element_type=jnp.float32)
              + bn1_ref[...])
    hnew_ref[...] = h + jnp.dot(u, wn2_ref[...],
                                preferred_element_type=jnp.float32) + bn2_ref[...]
    cnt = jnp.maximum(t[:, 3:4], 1.0)
    ynew_ref[...] = y_ref[...] + t[:, 0:4] * jnp.concatenate(
        [jnp.ones((1, 3), jnp.float32), jnp.zeros((1, 1), jnp.float32)],
        axis=1) / cnt


def _stage4(h, aggp, tp, y4, Wn1h, Wn1a, bn1, Wn2, bn2):
    grid = (N // NB,)
    return pl.pallas_call(
        _stage4_body,
        grid=grid,
        in_specs=[
            pl.BlockSpec((NB, D), lambda i: (i, 0)),
            pl.BlockSpec((2 * SL, NB, H), lambda i: (0, i, 0)),
            pl.BlockSpec((2 * SL, NB, 16), lambda i: (0, i, 0)),
            pl.BlockSpec((NB, 4), lambda i: (i, 0)),
            pl.BlockSpec((D, H), lambda i: (0, 0)),
            pl.BlockSpec((H, H), lambda i: (0, 0)),
            pl.BlockSpec((1, H), lambda i: (0, 0)),
            pl.BlockSpec((H, D), lambda i: (0, 0)),
            pl.BlockSpec((1, D), lambda i: (0, 0)),
        ],
        out_specs=[
            pl.BlockSpec((NB, D), lambda i: (i, 0)),
            pl.BlockSpec((NB, 4), lambda i: (i, 0)),
        ],
        out_shape=[
            jax.ShapeDtypeStruct((N, D), jnp.float32),
            jax.ShapeDtypeStruct((N, 4), jnp.float32),
        ],
    )(h, aggp, tp, y4, Wn1h, Wn1a, bn1, Wn2, bn2)


def kernel(h, edge_index, y, edge_attr, We1, be1, We2, be2,
           Wn1, bn1, Wn2, bn2, Wc1, bc1, Wc2):
    row = edge_index[0]
    col = edge_index[1]
    We1a = We1[:D]
    We1b = We1[D:2 * D]
    we1r = We1[2 * D]
    w1ea = We1[2 * D + 1:]

    P, Q = _stage0(h, We1a, We1b)

    y3 = y.T.reshape(3 * N)
    w1ea5 = jnp.concatenate([w1ea, we1r.reshape(1, H)], axis=0)

    aggps, tps = [], []
    for i in range(SL):
        sl = slice(i * ES, (i + 1) * ES)
        sa, sb, cd1 = _stage1(P, Q, y3, row[sl], col[sl])
        radial_row = cd1.reshape(ES, 4)[:, 3].reshape(1, ES)
        ea5t = jnp.concatenate([edge_attr[sl].T, radial_row], axis=0)
        feat, w2d = _stage2(sa, sb, ea5t, w1ea5, be1.reshape(1, H), We2,
                            be2.reshape(1, H), Wc1, bc1.reshape(1, H),
                            Wc2.reshape(1, H))
        aggp_flat, tp_flat = _stage3(feat, w2d.reshape(ES), cd1, row[sl])
        aggps.append(aggp_flat.reshape(2, N, H))
        tps.append(tp_flat.reshape(2, N, 16))

    y4 = jnp.pad(y, ((0, 0), (0, 1)))
    h_new, y_new4 = _stage4(h, jnp.concatenate(aggps, axis=0),
                            jnp.concatenate(tps, axis=0), y4,
                            Wn1[:D], Wn1[D:], bn1.reshape(1, H),
                            Wn2, bn2.reshape(1, D))
    return (h_new, y_new4[:, :3], edge_attr)


# speedup vs baseline: 1.6768x; 1.0218x over previous
import functools

import jax
import jax.numpy as jnp
from jax import lax
from jax.experimental import pallas as pl
from jax.experimental.pallas import tpu as pltpu
from jax.experimental.pallas import tpu_sc as plsc

N = 10000
E = 320000
D = 128
H = 128
SL = 4
ES = E // SL
EB = 3200
NB = 400

NC = 2
NS = 16
L = 16
NW = NC * NS
CE = 128
NCHUNKS = ES // CE
FULL_TRIPS = NCHUNKS // NW
EXTRA = NCHUNKS - FULL_TRIPS * NW


def _silu(x):
    return x * jax.nn.sigmoid(x)


def _stage0_body(h_ref, wa_ref, wb_ref, p_ref, q_ref):
    hb = h_ref[...]
    p_ref[...] = jnp.dot(hb, wa_ref[...], preferred_element_type=jnp.float32)
    q_ref[...] = jnp.dot(hb, wb_ref[...], preferred_element_type=jnp.float32)


def _stage0(h, We1a, We1b):
    grid = (N // NB,)
    return pl.pallas_call(
        _stage0_body,
        grid=grid,
        in_specs=[
            pl.BlockSpec((NB, D), lambda i: (i, 0)),
            pl.BlockSpec((D, H), lambda i: (0, 0)),
            pl.BlockSpec((D, H), lambda i: (0, 0)),
        ],
        out_specs=[
            pl.BlockSpec((NB, H), lambda i: (i, 0)),
            pl.BlockSpec((NB, H), lambda i: (i, 0)),
        ],
        out_shape=[
            jax.ShapeDtypeStruct((N, H), jnp.float32),
            jax.ShapeDtypeStruct((N, H), jnp.float32),
        ],
    )(h, We1a, We1b)


def _sc_mesh():
    return plsc.VectorSubcoreMesh(core_axis_name="c", subcore_axis_name="s",
                                  num_cores=NC, num_subcores=NS)


def _stage1(P, Q, y3, row, col):
    @functools.partial(
        pl.kernel, mesh=_sc_mesh(),
        compiler_params=pltpu.CompilerParams(needs_layout_passes=False,
                                             use_tc_tiling_on_sc=False),
        out_type=[jax.ShapeDtypeStruct((ES, H), jnp.float32),
                  jax.ShapeDtypeStruct((ES, H), jnp.float32),
                  jax.ShapeDtypeStruct((4 * ES,), jnp.float32)],
        scratch_types=[
            pltpu.VMEM((CE,), jnp.int32),
            pltpu.VMEM((CE,), jnp.int32),
            pltpu.VMEM((CE,), jnp.int32),
            pltpu.VMEM((CE,), jnp.int32),
            pltpu.VMEM((CE, H), jnp.float32),
            pltpu.VMEM((CE, H), jnp.float32),
            pltpu.VMEM((CE, H), jnp.float32),
            pltpu.VMEM((CE, H), jnp.float32),
            pltpu.VMEM((4 * CE,), jnp.float32),
            pltpu.VMEM((4 * CE,), jnp.float32),
            pltpu.VMEM((3 * N,), jnp.float32),
            pltpu.SemaphoreType.DMA,
            pltpu.SemaphoreType.DMA,
            pltpu.SemaphoreType.DMA,
            pltpu.SemaphoreType.DMA,
        ],
    )
    def k1(p_hbm, q_hbm, y_hbm, row_hbm, col_hbm, sa_hbm, sb_hbm, cd_hbm,
           rv0, rv1, cv0, cv1, pb0, pb1, qb0, qb1, cd0, cd1, ybuf,
           semg0, semg1, semo0, semo1):
        wid = lax.axis_index("s") * NC + lax.axis_index("c")
        pltpu.sync_copy(y_hbm, ybuf)
        rv = [rv0, rv1]
        cv = [cv0, cv1]
        pb = [pb0, pb1]
        qb = [qb0, qb1]
        cdb = [cd0, cd1]
        semg = [semg0, semg1]
        semo = [semo0, semo1]
        iota = lax.iota(jnp.int32, L)
        ntrips = FULL_TRIPS + jnp.where(wid < EXTRA, 1, 0)

        def ebase(ci):
            return (wid + ci * NW) * CE

        def fetch(ci, b):
            base = ebase(ci)
            pltpu.sync_copy(row_hbm.at[pl.ds(base, CE)], rv[b])
            pltpu.sync_copy(col_hbm.at[pl.ds(base, CE)], cv[b])
            pltpu.async_copy(p_hbm.at[rv[b]], pb[b], semg[b])
            pltpu.async_copy(q_hbm.at[cv[b]], qb[b], semg[b])

        def wait_gathers(b):
            pltpu.make_async_copy(p_hbm.at[rv[b]], pb[b], semg[b]).wait()
            pltpu.make_async_copy(q_hbm.at[cv[b]], qb[b], semg[b]).wait()

        def wait_outs(ci, b):
            base = ebase(ci)
            pltpu.make_async_copy(pb[b], sa_hbm.at[pl.ds(base, CE)],
                                  semo[b]).wait()
            pltpu.make_async_copy(qb[b], sb_hbm.at[pl.ds(base, CE)],
                                  semo[b]).wait()
            pltpu.make_async_copy(cdb[b], cd_hbm.at[pl.ds(base * 4, CE * 4)],
                                  semo[b]).wait()

        def compute(ci, b):
            for g in range(CE // L):
                e16 = iota + g * L
                r16 = rv[b][pl.ds(g * L, L)]
                c16 = cv[b][pl.ds(g * L, L)]
                dx = (plsc.load_gather(ybuf, [r16])
                      - plsc.load_gather(ybuf, [c16]))
                dy = (plsc.load_gather(ybuf, [r16 + N])
                      - plsc.load_gather(ybuf, [c16 + N]))
                dz = (plsc.load_gather(ybuf, [r16 + 2 * N])
                      - plsc.load_gather(ybuf, [c16 + 2 * N]))
                e4 = e16 * 4
                plsc.store_scatter(cdb[b], [e4], dx)
                plsc.store_scatter(cdb[b], [e4 + 1], dy)
                plsc.store_scatter(cdb[b], [e4 + 2], dz)
                plsc.store_scatter(cdb[b], [e4 + 3],
                                   dx * dx + dy * dy + dz * dz)

            wait_gathers(b)
            base = ebase(ci)
            pltpu.async_copy(pb[b], sa_hbm.at[pl.ds(base, CE)], semo[b])
            pltpu.async_copy(qb[b], sb_hbm.at[pl.ds(base, CE)], semo[b])
            pltpu.async_copy(cdb[b], cd_hbm.at[pl.ds(base * 4, CE * 4)],
                             semo[b])

        def iteration(ci, b):
            nb = 1 - b

            @pl.when(ci + 1 < ntrips)
            def _():
                @pl.when(ci >= 1)
                def _():
                    wait_outs(ci - 1, nb)

                fetch(ci + 1, nb)

            @pl.when(ci < ntrips)
            def _():
                compute(ci, b)

        @pl.when(ntrips >= 1)
        def _():
            fetch(0, 0)

        def pair(j, carry):
            iteration(2 * j, 0)
            iteration(2 * j + 1, 1)
            return carry

        lax.fori_loop(0, (FULL_TRIPS + 2) // 2, pair, 0)
        wait_outs(ntrips - 1, 0)
        wait_outs(ntrips - 1, 1)

    return k1(P, Q, y3, row, col)


NPT = 624


def _stage3(feat, w1d, cd1, row):
    @functools.partial(
        pl.kernel, mesh=_sc_mesh(),
        compiler_params=pltpu.CompilerParams(needs_layout_passes=False,
                                             use_tc_tiling_on_sc=False),
        out_type=[jax.ShapeDtypeStruct((2 * N, H), jnp.float32),
                  jax.ShapeDtypeStruct((2 * N, 16), jnp.float32)],
        scratch_types=[
            pltpu.VMEM((CE,), jnp.int32),
            pltpu.VMEM((CE,), jnp.int32),
            pltpu.VMEM((CE, H), jnp.float32),
            pltpu.VMEM((CE, H), jnp.float32),
            pltpu.VMEM((CE,), jnp.float32),
            pltpu.VMEM((CE,), jnp.float32),
            pltpu.VMEM((4 * CE,), jnp.float32),
            pltpu.VMEM((4 * CE,), jnp.float32),
            pltpu.VMEM((CE, 16), jnp.float32),
            pltpu.VMEM((CE, 16), jnp.float32),
            pltpu.VMEM_SHARED((N, H), jnp.float32),
            pltpu.VMEM_SHARED((N, 16), jnp.float32),
            pltpu.SemaphoreType.DMA,
            pltpu.SemaphoreType.DMA,
            pltpu.SemaphoreType.DMA,
            pltpu.SemaphoreType.DMA,
        ],
    )
    def k3(f_hbm, w_hbm, cd_hbm, row_hbm, aggp_hbm, tp_hbm,
           rv0, rv1, fb0, fb1, wb0, wb1, cb0, cb1, tb0, tb1, sh_agg, sh_t,
           semg0, semg1, semo0, semo1):
        cid = lax.axis_index("c")
        sid = lax.axis_index("s")
        wid = sid * NC + cid
        rv = [rv0, rv1]
        fb = [fb0, fb1]
        wbs = [wb0, wb1]
        cb = [cb0, cb1]
        tb = [tb0, tb1]
        semg = [semg0, semg1]
        semo = [semo0, semo1]
        iota = lax.iota(jnp.int32, L)
        zeros = jnp.zeros((L,), jnp.float32)
        ones = jnp.ones((L,), jnp.float32)
        czero = jnp.zeros((L,), jnp.int32)
        cone = jnp.full((L,), 1, jnp.int32)
        ctwo = jnp.full((L,), 2, jnp.int32)
        cthree = jnp.full((L,), 3, jnp.int32)

        def zrow(i, carry):
            for k in range(H // L):
                fb0[i, pl.ds(k * L, L)] = zeros
            tb0[i, :] = zeros
            tb1[i, :] = zeros
            return carry

        lax.fori_loop(0, CE, zrow, 0)

        r0 = sid * NPT
        for j in range(4):
            pltpu.sync_copy(fb0, sh_agg.at[pl.ds(r0 + j * CE, CE)])
            pltpu.sync_copy(tb0, sh_t.at[pl.ds(r0 + j * CE, CE)])
        pltpu.sync_copy(fb0.at[pl.ds(0, 112)],
                        sh_agg.at[pl.ds(r0 + 4 * CE, 112)])
        pltpu.sync_copy(tb0.at[pl.ds(0, 112)],
                        sh_t.at[pl.ds(r0 + 4 * CE, 112)])

        @pl.when(sid == NS - 1)
        def _():
            pltpu.sync_copy(fb0.at[pl.ds(0, 16)],
                            sh_agg.at[pl.ds(N - 16, 16)])
            pltpu.sync_copy(tb0.at[pl.ds(0, 16)],
                            sh_t.at[pl.ds(N - 16, 16)])

        plsc.subcore_barrier()

        ntrips = FULL_TRIPS + jnp.where(wid < EXTRA, 1, 0)

        def ebase(ci):
            return (wid + ci * NW) * CE

        def fetch(ci, b):
            base = ebase(ci)
            pltpu.sync_copy(row_hbm.at[pl.ds(base, CE)], rv[b])
            pltpu.async_copy(f_hbm.at[pl.ds(base, CE)], fb[b], semg[b])
            pltpu.async_copy(w_hbm.at[pl.ds(base, CE)], wbs[b], semg[b])
            pltpu.async_copy(cd_hbm.at[pl.ds(base * 4, CE * 4)], cb[b],
                             semg[b])

        def wait_ins(ci, b):
            base = ebase(ci)
            pltpu.make_async_copy(f_hbm.at[pl.ds(base, CE)], fb[b],
                                  semg[b]).wait()
            pltpu.make_async_copy(w_hbm.at[pl.ds(base, CE)], wbs[b],
                                  semg[b]).wait()
            pltpu.make_async_copy(cd_hbm.at[pl.ds(base * 4, CE * 4)], cb[b],
                                  semg[b]).wait()

        def wait_scatters(b):
            pltpu.make_async_copy(fb[b], sh_agg.at[rv[b]], semo[b]).wait()
            pltpu.make_async_copy(tb[b], sh_t.at[rv[b]], semo[b]).wait()

        def compute(ci, b):
            wait_ins(ci, b)
            for g in range(CE // L):
                sl = pl.ds(g * L, L)
                e16 = iota + g * L
                e4 = e16 * 4
                wv = wbs[b][sl]
                tx = plsc.load_gather(cb[b], [e4]) * wv
                ty = plsc.load_gather(cb[b], [e4 + 1]) * wv
                tz = plsc.load_gather(cb[b], [e4 + 2]) * wv
                plsc.store_scatter(tb[b], [e16, czero], tx)
                plsc.store_scatter(tb[b], [e16, cone], ty)
                plsc.store_scatter(tb[b], [e16, ctwo], tz)
                plsc.store_scatter(tb[b], [e16, cthree], ones)
            pltpu.async_copy(fb[b], sh_agg.at[rv[b]], semo[b], add=True)
            pltpu.async_copy(tb[b], sh_t.at[rv[b]], semo[b], add=True)

        def iteration(ci, b):
            nb = 1 - b

            @pl.when(ci + 1 < ntrips)
            def _():
                @pl.when(ci >= 1)
                def _():
                    wait_scatters(nb)

                fetch(ci + 1, nb)

            @pl.when(ci < ntrips)
            def _():
                compute(ci, b)

        @pl.when(ntrips >= 1)
        def _():
            fetch(0, 0)

        def pair(j, carry):
            iteration(2 * j, 0)
            iteration(2 * j + 1, 1)
            return carry

        lax.fori_loop(0, (FULL_TRIPS + 2) // 2, pair, 0)
        wait_scatters(0)
        wait_scatters(1)
        plsc.subcore_barrier()

        pltpu.sync_copy(sh_agg.at[pl.ds(r0, 624)],
                        aggp_hbm.at[pl.ds(cid * N + r0, 624)])
        pltpu.sync_copy(sh_t.at[pl.ds(r0, 624)],
                        tp_hbm.at[pl.ds(cid * N + r0, 624)])

        @pl.when(sid == NS - 1)
        def _():
            pltpu.sync_copy(sh_agg.at[pl.ds(N - 16, 16)],
                            aggp_hbm.at[pl.ds(cid * N + N - 16, 16)])
            pltpu.sync_copy(sh_t.at[pl.ds(N - 16, 16)],
                            tp_hbm.at[pl.ds(cid * N + N - 16, 16)])

    return k3(feat, w1d, cd1, row)


def _stage2_body(sa_ref, sb_ref, ea_ref, w1ea_ref, be1_ref, we2_ref, be2_ref,
                 wc1_ref, bc1_ref, wc2_ref, feat_ref, w_ref):
    s = sa_ref[...] + sb_ref[...]
    ea = ea_ref[...]
    m = s + lax.dot_general(ea, w1ea_ref[...], (((0,), (0,)), ((), ())),
                            preferred_element_type=jnp.float32)
    m = _silu(m + be1_ref[...])
    feat = _silu(jnp.dot(m, we2_ref[...], preferred_element_type=jnp.float32)
                 + be2_ref[...])
    feat_ref[...] = feat
    c = _silu(jnp.dot(feat, wc1_ref[...], preferred_element_type=jnp.float32)
              + bc1_ref[...])
    w = lax.dot_general(wc2_ref[...], c, (((1,), (1,)), ((), ())),
                        preferred_element_type=jnp.float32)
    w_ref[...] = w[None]


def _stage2(sa, sb, ea5t, w1ea5, be1, We2, be2, Wc1, bc1, wc2row):
    grid = (ES // EB,)
    return pl.pallas_call(
        _stage2_body,
        grid=grid,
        in_specs=[
            pl.BlockSpec((EB, H), lambda i: (i, 0)),
            pl.BlockSpec((EB, H), lambda i: (i, 0)),
            pl.BlockSpec((5, EB), lambda i: (0, i)),
            pl.BlockSpec((5, H), lambda i: (0, 0)),
            pl.BlockSpec((1, H), lambda i: (0, 0)),
            pl.BlockSpec((H, H), lambda i: (0, 0)),
            pl.BlockSpec((1, H), lambda i: (0, 0)),
            pl.BlockSpec((H, H), lambda i: (0, 0)),
            pl.BlockSpec((1, H), lambda i: (0, 0)),
            pl.BlockSpec((1, H), lambda i: (0, 0)),
        ],
        out_specs=[
            pl.BlockSpec((EB, H), lambda i: (i, 0)),
            pl.BlockSpec((1, 1, EB), lambda i: (i, 0, 0)),
        ],
        out_shape=[
            jax.ShapeDtypeStruct((ES, H), jnp.float32),
            jax.ShapeDtypeStruct((ES // EB, 1, EB), jnp.float32),
        ],
    )(sa, sb, ea5t, w1ea5, be1, We2, be2, Wc1, bc1, wc2row)


def _stage4_body(h_ref, aggp_ref, tp_ref, y_ref,
                 wn1h_ref, wn1a_ref, bn1_ref,
                 wn2_ref, bn2_ref, hnew_ref, ynew_ref):
    h = h_ref[...]
    agg = jnp.sum(aggp_ref[...], axis=0)
    t = jnp.sum(tp_ref[...], axis=0)
    u = _silu(jnp.dot(h, wn1h_ref[...], preferred_element_type=jnp.float32)
              + jnp.dot(agg, wn1a_ref[...], preferred_element_type=jnp.float32)
              + bn1_ref[...])
    hnew_ref[...] = h + jnp.dot(u, wn2_ref[...],
                                preferred_element_type=jnp.float32) + bn2_ref[...]
    cnt = jnp.maximum(t[:, 3:4], 1.0)
    ynew_ref[...] = y_ref[...] + t[:, 0:4] * jnp.concatenate(
        [jnp.ones((1, 3), jnp.float32), jnp.zeros((1, 1), jnp.float32)],
        axis=1) / cnt


def _stage4(h, aggp, tp, y4, Wn1h, Wn1a, bn1, Wn2, bn2):
    grid = (N // NB,)
    return pl.pallas_call(
        _stage4_body,
        grid=grid,
        in_specs=[
            pl.BlockSpec((NB, D), lambda i: (i, 0)),
            pl.BlockSpec((2 * SL, NB, H), lambda i: (0, i, 0)),
            pl.BlockSpec((2 * SL, NB, 16), lambda i: (0, i, 0)),
            pl.BlockSpec((NB, 4), lambda i: (i, 0)),
            pl.BlockSpec((D, H), lambda i: (0, 0)),
            pl.BlockSpec((H, H), lambda i: (0, 0)),
            pl.BlockSpec((1, H), lambda i: (0, 0)),
            pl.BlockSpec((H, D), lambda i: (0, 0)),
            pl.BlockSpec((1, D), lambda i: (0, 0)),
        ],
        out_specs=[
            pl.BlockSpec((NB, D), lambda i: (i, 0)),
            pl.BlockSpec((NB, 4), lambda i: (i, 0)),
        ],
        out_shape=[
            jax.ShapeDtypeStruct((N, D), jnp.float32),
            jax.ShapeDtypeStruct((N, 4), jnp.float32),
        ],
    )(h, aggp, tp, y4, Wn1h, Wn1a, bn1, Wn2, bn2)


def kernel(h, edge_index, y, edge_attr, We1, be1, We2, be2,
           Wn1, bn1, Wn2, bn2, Wc1, bc1, Wc2):
    row = edge_index[0]
    col = edge_index[1]
    We1a = We1[:D]
    We1b = We1[D:2 * D]
    we1r = We1[2 * D]
    w1ea = We1[2 * D + 1:]

    P, Q = _stage0(h, We1a, We1b)

    y3 = y.T.reshape(3 * N)
    w1ea5 = jnp.concatenate([w1ea, we1r.reshape(1, H)], axis=0)

    aggps, tps = [], []
    for i in range(SL):
        sl = slice(i * ES, (i + 1) * ES)
        sa, sb, cd1 = _stage1(P, Q, y3, row[sl], col[sl])
        radial_row = cd1.reshape(ES, 4)[:, 3].reshape(1, ES)
        ea5t = jnp.concatenate([edge_attr[sl].T, radial_row], axis=0)
        feat, w2d = _stage2(sa, sb, ea5t, w1ea5, be1.reshape(1, H), We2,
                            be2.reshape(1, H), Wc1, bc1.reshape(1, H),
                            Wc2.reshape(1, H))
        aggp_flat, tp_flat = _stage3(feat, w2d.reshape(ES), cd1, row[sl])
        aggps.append(aggp_flat.reshape(2, N, H))
        tps.append(tp_flat.reshape(2, N, 16))

    y4 = jnp.pad(y, ((0, 0), (0, 1)))
    h_new, y_new4 = _stage4(h, jnp.concatenate(aggps, axis=0),
                            jnp.concatenate(tps, axis=0), y4,
                            Wn1[:D], Wn1[D:], bn1.reshape(1, H),
                            Wn2, bn2.reshape(1, D))
    return (h_new, y_new4[:, :3], edge_attr)
